# baseline (speedup 1.0000x reference)
import jax
import jax.numpy as jnp
from jax import lax
from jax.experimental import pallas as pl
from jax.experimental.pallas import tpu as pltpu
from jax.experimental.pallas import tpu_sc as plsc

B, F, C, D = 4096, 100, 1000, 64
NC, NS, L = 2, 16, 16
NW = NC * NS
BPT = B // NW
NBANK = 4
PAIR = 2 * NBANK
PAIRS = BPT // PAIR
OFFS = (0, 16, 32, 48, 64, 80, 84)


def _body(emb_hbm, x_hbm, fi_hbm, out_hbm, xb_v, idx_v, fi_v, off_v, rows_v,
          gsem, osem0, osem1):
    wid = lax.axis_index("s") * NC + lax.axis_index("c")
    b_base = wid * BPT
    pltpu.sync_copy(fi_hbm, fi_v)
    for o in OFFS:
        sl = pl.ds(o, L)
        off_v[sl] = fi_v[sl] * C

    def wait_store(bank, osem):
        pltpu.make_async_copy(rows_v.at[bank], out_hbm.at[pl.ds(0, NBANK)],
                              osem).wait()

    def phase(bank, i0, b0, osem):
        cps = [pltpu.async_copy(emb_hbm.at[idx_v.at[i0 + j]],
                                rows_v.at[bank, j], gsem)
               for j in range(NBANK)]
        for cp in cps:
            cp.wait()
        pltpu.async_copy(rows_v.at[bank], out_hbm.at[pl.ds(b0, NBANK)], osem)

    def pair(t, carry):
        b0 = b_base + t * PAIR
        pltpu.sync_copy(x_hbm.at[pl.ds(b0, PAIR)], xb_v)
        for i in range(PAIR):
            for o in OFFS:
                sl = pl.ds(o, L)
                idx_v[i, sl] = xb_v[i, sl] + off_v[sl]

        @pl.when(t >= 1)
        def _():
            wait_store(0, osem0)
        phase(0, 0, b0, osem0)

        @pl.when(t >= 1)
        def _():
            wait_store(1, osem1)
        phase(1, NBANK, b0 + NBANK, osem1)
        return carry

    lax.fori_loop(0, PAIRS, pair, 0)
    wait_store(0, osem0)
    wait_store(1, osem1)


def kernel(x, embedding, feature_idx):
    emb2d = embedding.reshape(F * C, D)
    mesh = plsc.VectorSubcoreMesh(core_axis_name="c", subcore_axis_name="s")
    k = pl.kernel(
        _body,
        mesh=mesh,
        compiler_params=pltpu.CompilerParams(use_tc_tiling_on_sc=False),
        out_type=jax.ShapeDtypeStruct((B, F, D), jnp.float32),
        scratch_types=[
            pltpu.VMEM((PAIR, F), jnp.int32),
            pltpu.VMEM((PAIR, F), jnp.int32),
            pltpu.VMEM((F,), jnp.int32),
            pltpu.VMEM((F,), jnp.int32),
            pltpu.VMEM((2, NBANK, F, D), jnp.float32),
            pltpu.SemaphoreType.DMA,
            pltpu.SemaphoreType.DMA,
            pltpu.SemaphoreType.DMA,
        ],
    )
    return k(emb2d, x, feature_idx)


# speedup vs baseline: 1.0936x; 1.0936x over previous
import jax
import jax.numpy as jnp
from jax import lax
from jax.experimental import pallas as pl
from jax.experimental.pallas import tpu as pltpu
from jax.experimental.pallas import tpu_sc as plsc

B, F, C, D = 4096, 100, 1000, 64
PADW = 128
NC, NS, L = 2, 16, 16
NW = NC * NS
BPT = B // NW
GRP = 8
GRPS = BPT // GRP
RPI = 10
OFFS = (0, 16, 32, 48, 64, 80, 84)
DOFF = (0, 16, 32, 48)


def _body(emb_hbm, x_hbm, fi_hbm, out_hbm, xb_v, idx_v, fi_v, off_v,
          g_v, s_v, gsem0, gsem1, osem0, osem1):
    wid = lax.axis_index("s") * NC + lax.axis_index("c")
    b_base = wid * BPT
    pltpu.sync_copy(fi_hbm, fi_v)
    for o in OFFS:
        sl = pl.ds(o, L)
        off_v[sl] = fi_v[sl] * C

    gsems = (gsem0, gsem1)
    osems = (osem0, osem1)

    def fire_gather(i, p):
        return pltpu.async_copy(emb_hbm.at[idx_v.at[i]], g_v.at[p], gsems[p])

    def wait_store(p):
        pltpu.make_async_copy(s_v.at[p], out_hbm.at[0], osems[p]).wait()

    def compact(p):
        def rows(r0, carry):
            for rr in range(RPI):
                r = r0 * RPI + rr
                for o in DOFF:
                    sl = pl.ds(o, L)
                    s_v[p, r, sl] = g_v[p, r, sl]
            return carry
        lax.fori_loop(0, F // RPI, rows, 0)

    def group(t, carry):
        b0 = b_base + t * GRP
        pltpu.sync_copy(x_hbm.at[pl.ds(b0, GRP)], xb_v)
        for i in range(GRP):
            for o in OFFS:
                sl = pl.ds(o, L)
                idx_v[i, sl] = xb_v[i, sl] + off_v[sl]

        cps = [None] * GRP
        cps[0] = fire_gather(0, 0)
        for i in range(GRP):
            p = i % 2
            if i + 1 < GRP:
                cps[i + 1] = fire_gather(i + 1, (i + 1) % 2)
            cps[i].wait()
            if i < 2:
                @pl.when(t >= 1)
                def _():
                    wait_store(p)
            else:
                wait_store(p)
            compact(p)
            pltpu.async_copy(s_v.at[p], out_hbm.at[b0 + i], osems[p])
        return carry

    lax.fori_loop(0, GRPS, group, 0)
    wait_store(0)
    wait_store(1)


def kernel(x, embedding, feature_idx):
    emb_pad = jnp.pad(embedding.reshape(F * C, D), ((0, 0), (0, PADW - D)))
    mesh = plsc.VectorSubcoreMesh(core_axis_name="c", subcore_axis_name="s")
    k = pl.kernel(
        _body,
        mesh=mesh,
        compiler_params=pltpu.CompilerParams(use_tc_tiling_on_sc=True),
        out_type=jax.ShapeDtypeStruct((B, F, D), jnp.float32),
        scratch_types=[
            pltpu.VMEM((GRP, F), jnp.int32),
            pltpu.VMEM((GRP, F), jnp.int32),
            pltpu.VMEM((F,), jnp.int32),
            pltpu.VMEM((F,), jnp.int32),
            pltpu.VMEM((2, F, PADW), jnp.float32),
            pltpu.VMEM((2, F, D), jnp.float32),
            pltpu.SemaphoreType.DMA,
            pltpu.SemaphoreType.DMA,
            pltpu.SemaphoreType.DMA,
            pltpu.SemaphoreType.DMA,
        ],
    )
    return k(emb_pad, x, feature_idx)
